# Initial kernel scaffold; baseline (speedup 1.0000x reference)
import jax
import jax.numpy as jnp
from jax.experimental import pallas as pl


def kernel(h, edge_index, snorm_n, W_fc, W_attn):
    raise NotImplementedError("write your pallas kernel here")


# speedup vs baseline: 53.7111x; 53.7111x over previous
import jax
import jax.numpy as jnp
from jax import lax
from jax.experimental import pallas as pl
from jax.experimental.pallas import tpu as pltpu
from jax.experimental.pallas import tpu_sc as plsc

N = 100000
E = 3200000
IN_DIM = 128
OUT_DIM = 16

NC = 2
NS = 16
NW = NC * NS

CH = 128
CPB = 8
BLOCKS_PW = 98
CH_PW = CPB * BLOCKS_PW
E_PAD = NW * CH_PW * CH
TOTC = E_PAD // CH

ACC_ROWS = 100352
ROWS_PT = ACC_ROWS // NS
NPAD = 100096
WOUT = NPAD // NS
TRASH = ACC_ROWS - N

MM_BN = 2000


def _mm_body(h_ref, w_ref, z_ref):
    z_ref[...] = jnp.dot(h_ref[...], w_ref[...],
                         preferred_element_type=jnp.float32)


def _matmul(h, wt):
    return pl.pallas_call(
        _mm_body,
        grid=(N // MM_BN,),
        in_specs=[
            pl.BlockSpec((MM_BN, IN_DIM), lambda i: (i, 0)),
            pl.BlockSpec((IN_DIM, OUT_DIM), lambda i: (0, 0)),
        ],
        out_specs=pl.BlockSpec((MM_BN, OUT_DIM), lambda i: (i, 0)),
        out_shape=jax.ShapeDtypeStruct((N, OUT_DIM), jnp.float32),
    )(h, wt)


def _sc_body(z_hbm, src_hbm, dst_hbm, zrows_hbm, out_hbm,
             acc, sblk, dblk, rows, gsem):
    cid = lax.axis_index("c")
    sid = lax.axis_index("s")
    wid = sid * NC + cid

    pltpu.sync_copy(zrows_hbm, acc.at[pl.ds(sid * ROWS_PT, ROWS_PT)])
    plsc.subcore_barrier()

    base = wid * CH_PW

    def block(b, carry):
        row0 = base + b * CPB
        pltpu.sync_copy(src_hbm.at[pl.ds(row0, CPB)], sblk)
        pltpu.sync_copy(dst_hbm.at[pl.ds(row0, CPB)], dblk)
        copies = [pltpu.async_copy(z_hbm.at[sblk.at[c]], rows.at[c], gsem)
                  for c in range(CPB)]
        for c in range(CPB):
            copies[c].wait()
        for c in range(CPB):
            pltpu.sync_copy(rows.at[c], acc.at[dblk.at[c]], add=True)
        return carry

    lax.fori_loop(0, BLOCKS_PW, block, 0)
    plsc.subcore_barrier()

    pltpu.sync_copy(acc.at[pl.ds(sid * WOUT, WOUT)],
                    out_hbm.at[cid, pl.ds(sid * WOUT, WOUT)])


def _sc_scatter(z, src2d, dst2d, zrows):
    call = pl.kernel(
        _sc_body,
        out_type=jax.ShapeDtypeStruct((NC, NPAD, OUT_DIM), jnp.float32),
        mesh=plsc.VectorSubcoreMesh(core_axis_name="c", subcore_axis_name="s",
                                    num_cores=NC, num_subcores=NS),
        scratch_types=[
            pltpu.VMEM_SHARED((ACC_ROWS, OUT_DIM), jnp.float32),
            pltpu.VMEM((CPB, CH), jnp.int32),
            pltpu.VMEM((CPB, CH), jnp.int32),
            pltpu.VMEM((CPB, CH, OUT_DIM), jnp.float32),
            pltpu.SemaphoreType.DMA,
        ],
        compiler_params=pltpu.CompilerParams(use_tc_tiling_on_sc=False),
    )
    return call(z, src2d, dst2d, zrows)


def _fin_body(a_ref, b_ref, s_ref, o_ref):
    o_ref[...] = jnp.maximum((a_ref[0] + b_ref[0]) * s_ref[...], 0.0)


def _finalize(partials, snorm_n):
    return pl.pallas_call(
        _fin_body,
        grid=(N // MM_BN,),
        in_specs=[
            pl.BlockSpec((1, MM_BN, OUT_DIM), lambda i: (0, i, 0)),
            pl.BlockSpec((1, MM_BN, OUT_DIM), lambda i: (1, i, 0)),
            pl.BlockSpec((MM_BN, 1), lambda i: (i, 0)),
        ],
        out_specs=pl.BlockSpec((MM_BN, OUT_DIM), lambda i: (i, 0)),
        out_shape=jax.ShapeDtypeStruct((N, OUT_DIM), jnp.float32),
    )(partials, partials, snorm_n)


def kernel(h, edge_index, snorm_n, W_fc, W_attn):
    z = _matmul(h, W_fc.T)
    pad = E_PAD - E
    idx = jnp.arange(pad, dtype=jnp.int32)
    src_p = jnp.concatenate([edge_index[0], idx % N])
    dst_p = jnp.concatenate([edge_index[1], N + idx % TRASH])
    src2d = src_p.reshape(TOTC, CH)
    dst2d = dst_p.reshape(TOTC, CH)
    zrows = jnp.zeros((ROWS_PT, OUT_DIM), jnp.float32)
    partials = _sc_scatter(z, src2d, dst2d, zrows)
    return _finalize(partials, snorm_n)


# speedup vs baseline: 69.4043x; 1.2922x over previous
import jax
import jax.numpy as jnp
from jax import lax
from jax.experimental import pallas as pl
from jax.experimental.pallas import tpu as pltpu
from jax.experimental.pallas import tpu_sc as plsc

N = 100000
E = 3200000
IN_DIM = 128
OUT_DIM = 16

NC = 2
NS = 16
NW = NC * NS

CH = 128
CPB = 6
BLOCKS_PW = 131
CH_PW = CPB * BLOCKS_PW
E_PAD = NW * CH_PW * CH
TOTC = E_PAD // CH

ACC_ROWS = 100096
ROWS_PT = ACC_ROWS // NS
NPAD = 100096
WOUT = NPAD // NS
TRASH = ACC_ROWS - N

MM_BN = 2000


def _mm_body(h_ref, w_ref, z_ref):
    z_ref[...] = jnp.dot(h_ref[...], w_ref[...],
                         preferred_element_type=jnp.float32)


def _matmul(h, wt):
    return pl.pallas_call(
        _mm_body,
        grid=(N // MM_BN,),
        in_specs=[
            pl.BlockSpec((MM_BN, IN_DIM), lambda i: (i, 0)),
            pl.BlockSpec((IN_DIM, OUT_DIM), lambda i: (0, 0)),
        ],
        out_specs=pl.BlockSpec((MM_BN, OUT_DIM), lambda i: (i, 0)),
        out_shape=jax.ShapeDtypeStruct((N, OUT_DIM), jnp.float32),
    )(h, wt)


def _sc_body(z_hbm, src_hbm, dst_hbm, zrows_hbm, out_hbm,
             acc, sblk, dblk, rows, gsem, ssem):
    cid = lax.axis_index("c")
    sid = lax.axis_index("s")
    wid = sid * NC + cid

    pltpu.sync_copy(zrows_hbm, acc.at[pl.ds(sid * ROWS_PT, ROWS_PT)])
    plsc.subcore_barrier()

    base = wid * CH_PW

    def load_and_fire(blk, par):
        row0 = base + blk * CPB
        pltpu.sync_copy(src_hbm.at[pl.ds(row0, CPB)], sblk.at[par])
        pltpu.sync_copy(dst_hbm.at[pl.ds(row0, CPB)], dblk.at[par])
        for c in range(CPB):
            pltpu.async_copy(z_hbm.at[sblk.at[par, c]], rows.at[par, c], gsem)

    def drain_gathers(par):
        for c in range(CPB):
            pltpu.make_async_copy(z_hbm.at[sblk.at[par, c]],
                                  rows.at[par, c], gsem).wait()

    load_and_fire(0, 0)

    def block(b, carry):
        par = b % 2
        load_and_fire(b + 1, 1 - par)
        drain_gathers(par)
        for c in range(CPB):
            pltpu.async_copy(rows.at[par, c], acc.at[dblk.at[par, c]],
                             ssem, add=True)
        for c in range(CPB):
            pltpu.make_async_copy(rows.at[par, c],
                                  acc.at[dblk.at[par, c]], ssem).wait()
        return carry

    lax.fori_loop(0, BLOCKS_PW - 1, block, 0)

    lpar = (BLOCKS_PW - 1) % 2
    drain_gathers(lpar)
    for c in range(CPB):
        pltpu.sync_copy(rows.at[lpar, c], acc.at[dblk.at[lpar, c]], add=True)
    plsc.subcore_barrier()

    pltpu.sync_copy(acc.at[pl.ds(sid * WOUT, WOUT)],
                    out_hbm.at[cid, pl.ds(sid * WOUT, WOUT)])


def _sc_scatter(z, src2d, dst2d, zrows):
    call = pl.kernel(
        _sc_body,
        out_type=jax.ShapeDtypeStruct((NC, NPAD, OUT_DIM), jnp.float32),
        mesh=plsc.VectorSubcoreMesh(core_axis_name="c", subcore_axis_name="s",
                                    num_cores=NC, num_subcores=NS),
        scratch_types=[
            pltpu.VMEM_SHARED((ACC_ROWS, OUT_DIM), jnp.float32),
            pltpu.VMEM((2, CPB, CH), jnp.int32),
            pltpu.VMEM((2, CPB, CH), jnp.int32),
            pltpu.VMEM((2, CPB, CH, OUT_DIM), jnp.float32),
            pltpu.SemaphoreType.DMA,
            pltpu.SemaphoreType.DMA,
        ],
        compiler_params=pltpu.CompilerParams(use_tc_tiling_on_sc=False),
    )
    return call(z, src2d, dst2d, zrows)


def _fin_body(a_ref, b_ref, s_ref, o_ref):
    o_ref[...] = jnp.maximum((a_ref[0] + b_ref[0]) * s_ref[...], 0.0)


def _finalize(partials, snorm_n):
    return pl.pallas_call(
        _fin_body,
        grid=(N // MM_BN,),
        in_specs=[
            pl.BlockSpec((1, MM_BN, OUT_DIM), lambda i: (0, i, 0)),
            pl.BlockSpec((1, MM_BN, OUT_DIM), lambda i: (1, i, 0)),
            pl.BlockSpec((MM_BN, 1), lambda i: (i, 0)),
        ],
        out_specs=pl.BlockSpec((MM_BN, OUT_DIM), lambda i: (i, 0)),
        out_shape=jax.ShapeDtypeStruct((N, OUT_DIM), jnp.float32),
    )(partials, partials, snorm_n)


def kernel(h, edge_index, snorm_n, W_fc, W_attn):
    z = _matmul(h, W_fc.T)
    pad = E_PAD - E
    idx = jnp.arange(pad, dtype=jnp.int32)
    src_p = jnp.concatenate([edge_index[0], idx % N])
    dst_p = jnp.concatenate([edge_index[1], N + idx % TRASH])
    src2d = src_p.reshape(TOTC, CH)
    dst2d = dst_p.reshape(TOTC, CH)
    zrows = jnp.zeros((ROWS_PT, OUT_DIM), jnp.float32)
    partials = _sc_scatter(z, src2d, dst2d, zrows)
    return _finalize(partials, snorm_n)


# speedup vs baseline: 87.4024x; 1.2593x over previous
import jax
import jax.numpy as jnp
from jax import lax
from jax.experimental import pallas as pl
from jax.experimental.pallas import tpu as pltpu
from jax.experimental.pallas import tpu_sc as plsc

N = 100000
E = 3200000
IN_DIM = 128
OUT_DIM = 16

NC = 2
NS = 16
NW = NC * NS

CH = 128
CPB = 6
TOTC = E // CH
NB = 130
CH_PW = CPB * NB
NTAILC = TOTC - NW * CH_PW

ACC_ROWS = 100096
ROWS_PT = ACC_ROWS // NS
NPAD = 100096
WOUT = NPAD // NS

MM_BN = 5000

BF = 128
NFULL = N // BF
NTAILR = N - NFULL * BF
CPW_B = 25


def _mm_body(h_ref, w_ref, z_ref):
    z_ref[...] = jnp.dot(h_ref[...], w_ref[...],
                         preferred_element_type=jnp.float32)


def _matmul(h, wt):
    return pl.pallas_call(
        _mm_body,
        grid=(N // MM_BN,),
        in_specs=[
            pl.BlockSpec((MM_BN, IN_DIM), lambda i: (i, 0)),
            pl.BlockSpec((IN_DIM, OUT_DIM), lambda i: (0, 0)),
        ],
        out_specs=pl.BlockSpec((MM_BN, OUT_DIM), lambda i: (i, 0)),
        out_shape=jax.ShapeDtypeStruct((N, OUT_DIM), jnp.float32),
    )(h, wt)


def _sc_body(z_hbm, e_hbm, zrows_hbm, out_hbm,
             acc, eblk, rows, gsem, ssem, isem):
    cid = lax.axis_index("c")
    sid = lax.axis_index("s")
    wid = sid * NC + cid

    pltpu.sync_copy(zrows_hbm, acc.at[pl.ds(sid * ROWS_PT, ROWS_PT)])
    plsc.subcore_barrier()

    base = wid * CH_PW

    def load_idx_sync(blk, par):
        row0 = base + blk * CPB
        pltpu.sync_copy(e_hbm.at[0, pl.ds(row0, CPB)], eblk.at[par, 0])
        pltpu.sync_copy(e_hbm.at[1, pl.ds(row0, CPB)], eblk.at[par, 1])

    def load_idx_async(blk, par):
        row0 = base + blk * CPB
        pltpu.async_copy(e_hbm.at[0, pl.ds(row0, CPB)], eblk.at[par, 0], isem)
        pltpu.async_copy(e_hbm.at[1, pl.ds(row0, CPB)], eblk.at[par, 1], isem)

    def drain_idx(par):
        pltpu.make_async_copy(e_hbm.at[0, pl.ds(0, CPB)],
                              eblk.at[par, 0], isem).wait()
        pltpu.make_async_copy(e_hbm.at[1, pl.ds(0, CPB)],
                              eblk.at[par, 1], isem).wait()

    def fire_gathers(par):
        for c in range(CPB):
            pltpu.async_copy(z_hbm.at[eblk.at[par, 0, c]],
                             rows.at[par, pl.ds(c * CH, CH)], gsem)

    def drain_bulk(sem, par):
        pltpu.make_async_copy(out_hbm.at[0, pl.ds(0, CPB * CH)],
                              rows.at[par], sem).wait()

    def fire_scatters(par):
        for c in range(CPB):
            pltpu.async_copy(rows.at[par, pl.ds(c * CH, CH)],
                             acc.at[eblk.at[par, 1, c]], ssem, add=True)

    load_idx_sync(0, 0)
    fire_gathers(0)
    load_idx_async(1, 1)

    def block(b, carry):
        par = b % 2
        drain_idx(1 - par)
        fire_gathers(1 - par)
        drain_bulk(gsem, par)
        fire_scatters(par)
        drain_bulk(ssem, par)
        load_idx_async(b + 2, par)
        return carry

    lax.fori_loop(0, NB - 1, block, 0)

    lpar = (NB - 1) % 2
    drain_idx(1 - lpar)
    drain_bulk(gsem, lpar)
    fire_scatters(lpar)
    drain_bulk(ssem, lpar)

    def tail_chunk(chunk):
        pltpu.sync_copy(e_hbm.at[0, pl.ds(chunk, 1)], eblk.at[0, 0, pl.ds(0, 1)])
        pltpu.sync_copy(e_hbm.at[1, pl.ds(chunk, 1)], eblk.at[0, 1, pl.ds(0, 1)])
        pltpu.async_copy(z_hbm.at[eblk.at[0, 0, 0]],
                         rows.at[0, pl.ds(0, CH)], gsem).wait()
        pltpu.sync_copy(rows.at[0, pl.ds(0, CH)],
                        acc.at[eblk.at[0, 1, 0]], add=True)

    tail_chunk(NW * CH_PW + wid)

    @pl.when(wid < NTAILC - NW)
    def _():
        tail_chunk(NW * CH_PW + NW + wid)

    plsc.subcore_barrier()

    pltpu.sync_copy(acc.at[pl.ds(sid * WOUT, WOUT)],
                    out_hbm.at[cid, pl.ds(sid * WOUT, WOUT)])


def _sc_scatter(z, e3, zrows):
    call = pl.kernel(
        _sc_body,
        out_type=jax.ShapeDtypeStruct((NC, NPAD, OUT_DIM), jnp.float32),
        mesh=plsc.VectorSubcoreMesh(core_axis_name="c", subcore_axis_name="s",
                                    num_cores=NC, num_subcores=NS),
        scratch_types=[
            pltpu.VMEM_SHARED((ACC_ROWS, OUT_DIM), jnp.float32),
            pltpu.VMEM((2, 2, CPB, CH), jnp.int32),
            pltpu.VMEM((2, CPB * CH, OUT_DIM), jnp.float32),
            pltpu.SemaphoreType.DMA,
            pltpu.SemaphoreType.DMA,
            pltpu.SemaphoreType.DMA,
        ],
        compiler_params=pltpu.CompilerParams(use_tc_tiling_on_sc=False),
    )
    return call(z, e3, zrows)


def _bcast_lane(vec, lane):
    idx = jnp.full((16, 1), lane, jnp.int32)
    return lax.gather(
        vec, idx,
        dimension_numbers=lax.GatherDimensionNumbers(
            offset_dims=(), collapsed_slice_dims=(0,), start_index_map=(0,)),
        slice_sizes=(1,),
        mode=lax.GatherScatterMode.PROMISE_IN_BOUNDS)


def _fin_body(p_hbm, sn_hbm, out_hbm, v0, v1, sv, ov):
    cid = lax.axis_index("c")
    sid = lax.axis_index("s")
    wid = sid * NC + cid

    def do_chunk(chunk, nrows):
        r0 = chunk * BF
        pltpu.sync_copy(p_hbm.at[0, pl.ds(r0, nrows)], v0.at[pl.ds(0, nrows)])
        pltpu.sync_copy(p_hbm.at[1, pl.ds(r0, nrows)], v1.at[pl.ds(0, nrows)])
        pltpu.sync_copy(sn_hbm.at[pl.ds(r0, nrows)], sv.at[pl.ds(0, nrows)])
        for g in range(nrows // 16):
            s16 = sv[pl.ds(g * 16, 16)]
            for r in range(16):
                row = g * 16 + r
                sr = _bcast_lane(s16, r)
                ov[row, :] = jnp.maximum(v0[row, :] + v1[row, :], 0.0) * sr
        pltpu.sync_copy(ov.at[pl.ds(0, nrows)], out_hbm.at[pl.ds(r0, nrows)])

    def loop_body(j, carry):
        chunk = wid + NW * j

        @pl.when(chunk < NFULL)
        def _():
            do_chunk(chunk, BF)

        return carry

    lax.fori_loop(0, CPW_B, loop_body, 0)

    @pl.when(wid == 0)
    def _():
        do_chunk(NFULL, NTAILR)


def _finalize(partials, snorm_flat):
    call = pl.kernel(
        _fin_body,
        out_type=jax.ShapeDtypeStruct((N, OUT_DIM), jnp.float32),
        mesh=plsc.VectorSubcoreMesh(core_axis_name="c", subcore_axis_name="s",
                                    num_cores=NC, num_subcores=NS),
        scratch_types=[
            pltpu.VMEM((BF, OUT_DIM), jnp.float32),
            pltpu.VMEM((BF, OUT_DIM), jnp.float32),
            pltpu.VMEM((BF,), jnp.float32),
            pltpu.VMEM((BF, OUT_DIM), jnp.float32),
        ],
        compiler_params=pltpu.CompilerParams(use_tc_tiling_on_sc=False),
    )
    return call(partials, snorm_flat)


def kernel(h, edge_index, snorm_n, W_fc, W_attn):
    z = _matmul(h, W_fc.T)
    e3 = edge_index.reshape(2, TOTC, CH)
    zrows = jnp.zeros((ROWS_PT, OUT_DIM), jnp.float32)
    partials = _sc_scatter(z, e3, zrows)
    return _finalize(partials, snorm_n.reshape(N))


# speedup vs baseline: 95.5210x; 1.0929x over previous
import jax
import jax.numpy as jnp
from jax import lax
from jax.experimental import pallas as pl
from jax.experimental.pallas import tpu as pltpu
from jax.experimental.pallas import tpu_sc as plsc

N = 100000
E = 3200000
IN_DIM = 128
OUT_DIM = 16

NC = 2
NS = 16
NW = NC * NS

CH = 128
CPB = 6
TOTC = E // CH
NB = 130
CH_PW = CPB * NB
NTAILC = TOTC - NW * CH_PW

ACC_ROWS = 100096
ROWS_PT = ACC_ROWS // NS
NPAD = 100096
WOUT = NPAD // NS

MM_BN = 10000

BF = 400
NFULL = N // BF
CPW_B = 8


def _mm_body(h_ref, w_ref, s_ref, z_ref, sf_ref):
    z_ref[...] = jnp.dot(h_ref[...], w_ref[...],
                         preferred_element_type=jnp.float32)
    sf_ref[...] = s_ref[...].reshape(1, 1, MM_BN)


def _matmul(h, wt, snorm_n):
    nb = N // MM_BN
    return pl.pallas_call(
        _mm_body,
        grid=(nb,),
        in_specs=[
            pl.BlockSpec((MM_BN, IN_DIM), lambda i: (i, 0)),
            pl.BlockSpec((IN_DIM, OUT_DIM), lambda i: (0, 0)),
            pl.BlockSpec((MM_BN, 1), lambda i: (i, 0)),
        ],
        out_specs=[
            pl.BlockSpec((MM_BN, OUT_DIM), lambda i: (i, 0)),
            pl.BlockSpec((1, 1, MM_BN), lambda i: (i, 0, 0)),
        ],
        out_shape=[
            jax.ShapeDtypeStruct((N, OUT_DIM), jnp.float32),
            jax.ShapeDtypeStruct((nb, 1, MM_BN), jnp.float32),
        ],
    )(h, wt, snorm_n)


def _sc_body(z_hbm, e_hbm, zrows_hbm, out_hbm,
             acc, eblk, rows, gsem, ssem, isem):
    cid = lax.axis_index("c")
    sid = lax.axis_index("s")
    wid = sid * NC + cid

    pltpu.sync_copy(zrows_hbm, acc.at[pl.ds(sid * ROWS_PT, ROWS_PT)])
    plsc.subcore_barrier()

    base = wid * CH_PW

    def load_idx_sync(blk, par):
        row0 = base + blk * CPB
        pltpu.sync_copy(e_hbm.at[:, pl.ds(row0, CPB)], eblk.at[par])

    def load_idx_async(blk, par):
        row0 = base + blk * CPB
        pltpu.async_copy(e_hbm.at[:, pl.ds(row0, CPB)], eblk.at[par], isem)

    def drain_idx(par):
        pltpu.make_async_copy(e_hbm.at[:, pl.ds(0, CPB)],
                              eblk.at[par], isem).wait()

    def fire_gathers(slot, rb):
        for c in range(CPB):
            pltpu.async_copy(z_hbm.at[eblk.at[slot, 0, c]],
                             rows.at[rb, pl.ds(c * CH, CH)], gsem)

    def drain_bulk(sem, rb):
        pltpu.make_async_copy(out_hbm.at[0, pl.ds(0, CPB * CH)],
                              rows.at[rb], sem).wait()

    def fire_scatters(slot, rb):
        for c in range(CPB):
            pltpu.async_copy(rows.at[rb, pl.ds(c * CH, CH)],
                             acc.at[eblk.at[slot, 1, c]], ssem, add=True)

    load_idx_sync(0, 0)
    fire_gathers(0, 0)
    load_idx_async(1, 1)

    def block(b, carry):
        par = b % 2
        drain_idx((b + 1) % 3)

        @pl.when(b >= 1)
        def _():
            drain_bulk(ssem, 1 - par)

        fire_gathers((b + 1) % 3, 1 - par)
        drain_bulk(gsem, par)
        fire_scatters(b % 3, par)
        load_idx_async(b + 2, (b + 2) % 3)
        return carry

    lax.fori_loop(0, NB - 1, block, 0)

    lpar = (NB - 1) % 2
    drain_idx(NB % 3)
    drain_bulk(ssem, 1 - lpar)
    drain_bulk(gsem, lpar)
    fire_scatters((NB - 1) % 3, lpar)
    drain_bulk(ssem, lpar)

    def tail_chunk(chunk):
        pltpu.sync_copy(e_hbm.at[0, pl.ds(chunk, 1)], eblk.at[0, 0, pl.ds(0, 1)])
        pltpu.sync_copy(e_hbm.at[1, pl.ds(chunk, 1)], eblk.at[0, 1, pl.ds(0, 1)])
        pltpu.async_copy(z_hbm.at[eblk.at[0, 0, 0]],
                         rows.at[0, pl.ds(0, CH)], gsem).wait()
        pltpu.sync_copy(rows.at[0, pl.ds(0, CH)],
                        acc.at[eblk.at[0, 1, 0]], add=True)

    tail_chunk(NW * CH_PW + wid)

    @pl.when(wid < NTAILC - NW)
    def _():
        tail_chunk(NW * CH_PW + NW + wid)

    plsc.subcore_barrier()

    pltpu.sync_copy(acc.at[pl.ds(sid * WOUT, WOUT)],
                    out_hbm.at[cid, pl.ds(sid * WOUT, WOUT)])


def _sc_scatter(z, e3, zrows):
    call = pl.kernel(
        _sc_body,
        out_type=jax.ShapeDtypeStruct((NC, NPAD, OUT_DIM), jnp.float32),
        mesh=plsc.VectorSubcoreMesh(core_axis_name="c", subcore_axis_name="s",
                                    num_cores=NC, num_subcores=NS),
        scratch_types=[
            pltpu.VMEM_SHARED((ACC_ROWS, OUT_DIM), jnp.float32),
            pltpu.VMEM((3, 2, CPB, CH), jnp.int32),
            pltpu.VMEM((2, CPB * CH, OUT_DIM), jnp.float32),
            pltpu.SemaphoreType.DMA,
            pltpu.SemaphoreType.DMA,
            pltpu.SemaphoreType.DMA,
        ],
        compiler_params=pltpu.CompilerParams(use_tc_tiling_on_sc=False),
    )
    return call(z, e3, zrows)


def _bcast_lane(vec, lane):
    idx = jnp.full((16, 1), lane, jnp.int32)
    return lax.gather(
        vec, idx,
        dimension_numbers=lax.GatherDimensionNumbers(
            offset_dims=(), collapsed_slice_dims=(0,), start_index_map=(0,)),
        slice_sizes=(1,),
        mode=lax.GatherScatterMode.PROMISE_IN_BOUNDS)


def _fin_body(p_hbm, sn_hbm, out_hbm, vp, sv, ov, lsem, osem):
    cid = lax.axis_index("c")
    sid = lax.axis_index("s")
    wid = sid * NC + cid

    spr = MM_BN // BF

    def fire_loads(j, buf):
        cj = wid + NW * j

        @pl.when(cj < NFULL)
        def _():
            r0 = cj * BF
            pltpu.async_copy(p_hbm.at[:, pl.ds(r0, BF)], vp.at[buf], lsem)
            pltpu.async_copy(sn_hbm.at[cj // spr, 0, pl.ds((cj % spr) * BF, BF)],
                             sv.at[buf], lsem)

    def drain_loads(j, buf):
        cj = wid + NW * j

        @pl.when(cj < NFULL)
        def _():
            pltpu.make_async_copy(p_hbm.at[:, pl.ds(0, BF)],
                                  vp.at[buf], lsem).wait()
            pltpu.make_async_copy(sn_hbm.at[0, 0, pl.ds(0, BF)],
                                  sv.at[buf], lsem).wait()

    def compute(buf, nrows):
        def grp(g, carry):
            s16 = sv[buf, pl.ds(g * 16, 16)]
            for r in range(16):
                row = g * 16 + r
                sr = _bcast_lane(s16, r)
                ov[buf, row, :] = jnp.maximum(
                    vp[buf, 0, row, :] + vp[buf, 1, row, :], 0.0) * sr
            return carry

        lax.fori_loop(0, nrows // 16, grp, 0)

    def drain_store(j, buf):
        cj = wid + NW * j

        @pl.when(cj < NFULL)
        def _():
            pltpu.make_async_copy(ov.at[buf],
                                  out_hbm.at[pl.ds(0, BF)], osem).wait()

    fire_loads(0, 0)

    def loop_body(j, carry):
        buf = j % 2
        fire_loads(j + 1, 1 - buf)
        drain_loads(j, buf)

        @pl.when(j >= 2)
        def _():
            drain_store(j - 2, buf)

        cj = wid + NW * j

        @pl.when(cj < NFULL)
        def _():
            compute(buf, BF)
            pltpu.async_copy(ov.at[buf], out_hbm.at[pl.ds(cj * BF, BF)], osem)

        return carry

    lax.fori_loop(0, CPW_B, loop_body, 0)
    drain_store(CPW_B - 2, CPW_B % 2)
    drain_store(CPW_B - 1, (CPW_B - 1) % 2)


def _finalize(partials, snorm_n):
    call = pl.kernel(
        _fin_body,
        out_type=jax.ShapeDtypeStruct((N, OUT_DIM), jnp.float32),
        mesh=plsc.VectorSubcoreMesh(core_axis_name="c", subcore_axis_name="s",
                                    num_cores=NC, num_subcores=NS),
        scratch_types=[
            pltpu.VMEM((2, 2, BF, OUT_DIM), jnp.float32),
            pltpu.VMEM((2, BF), jnp.float32),
            pltpu.VMEM((2, BF, OUT_DIM), jnp.float32),
            pltpu.SemaphoreType.DMA,
            pltpu.SemaphoreType.DMA,
        ],
        compiler_params=pltpu.CompilerParams(use_tc_tiling_on_sc=False),
    )
    return call(partials, snorm_n)


def kernel(h, edge_index, snorm_n, W_fc, W_attn):
    z, snf = _matmul(h, W_fc.T, snorm_n)
    e3 = edge_index.reshape(2, TOTC, CH)
    zrows = jnp.zeros((ROWS_PT, OUT_DIM), jnp.float32)
    partials = _sc_scatter(z, e3, zrows)
    return _finalize(partials, snf)


# speedup vs baseline: 100.8037x; 1.0553x over previous
import jax
import jax.numpy as jnp
from jax import lax
from jax.experimental import pallas as pl
from jax.experimental.pallas import tpu as pltpu
from jax.experimental.pallas import tpu_sc as plsc

N = 100000
E = 3200000
IN_DIM = 128
OUT_DIM = 16

NC = 2
NS = 16
NW = NC * NS

CH = 128
CPB = 6
TOTC = E // CH
NB = 130
CH_PW = CPB * NB
NTAILC = TOTC - NW * CH_PW

ACC_ROWS = 100096
ROWS_PT = ACC_ROWS // NS
NPAD = 100096
WOUT = NPAD // NS

MM_BN = 10000

BF = 400
NFULL = N // BF
CPW_B = 8


def _mm_body(h_ref, w_ref, z_ref):
    z_ref[...] = jnp.dot(h_ref[...], w_ref[...],
                         preferred_element_type=jnp.float32)


def _matmul(h, wt):
    return pl.pallas_call(
        _mm_body,
        grid=(N // MM_BN,),
        in_specs=[
            pl.BlockSpec((MM_BN, IN_DIM), lambda i: (i, 0)),
            pl.BlockSpec((IN_DIM, OUT_DIM), lambda i: (0, 0)),
        ],
        out_specs=pl.BlockSpec((MM_BN, OUT_DIM), lambda i: (i, 0)),
        out_shape=jax.ShapeDtypeStruct((N, OUT_DIM), jnp.float32),
    )(h, wt)


def _snf_body(s_ref, o_ref):
    o_ref[...] = s_ref[...].reshape(1, 1, MM_BN)


def _snf(snorm_n):
    nb = N // MM_BN
    return pl.pallas_call(
        _snf_body,
        grid=(nb,),
        in_specs=[pl.BlockSpec((MM_BN, 1), lambda i: (i, 0))],
        out_specs=pl.BlockSpec((1, 1, MM_BN), lambda i: (i, 0, 0)),
        out_shape=jax.ShapeDtypeStruct((nb, 1, MM_BN), jnp.float32),
    )(snorm_n)


def _sc_body(z_hbm, e_hbm, zrows_hbm, out_hbm,
             acc, sflat, dblk, rows, gsem, ssem, isem):
    cid = lax.axis_index("c")
    sid = lax.axis_index("s")
    wid = sid * NC + cid

    pltpu.sync_copy(zrows_hbm, acc.at[pl.ds(sid * ROWS_PT, ROWS_PT)])
    plsc.subcore_barrier()

    base = wid * CH_PW * CH

    def load_idx(blk, slot, copy_fn):
        off = base + blk * CPB * CH
        copy_fn(e_hbm.at[0, pl.ds(off, CPB * CH)], sflat.at[slot])
        for c in range(CPB):
            copy_fn(e_hbm.at[1, pl.ds(off + c * CH, CH)], dblk.at[slot, c])

    def load_idx_sync(blk, slot):
        load_idx(blk, slot, pltpu.sync_copy)

    def load_idx_async(blk, slot):
        load_idx(blk, slot,
                 lambda s, d: pltpu.async_copy(s, d, isem))

    def drain_idx(slot):
        for _ in range(2):
            pltpu.make_async_copy(e_hbm.at[0, pl.ds(0, CPB * CH)],
                                  sflat.at[slot], isem).wait()

    def fire_gathers(slot, rb):
        for c in range(CPB):
            pltpu.async_copy(z_hbm.at[sflat.at[slot, pl.ds(c * CH, CH)]],
                             rows.at[rb, pl.ds(c * CH, CH)], gsem)

    def drain_bulk(sem, rb):
        pltpu.make_async_copy(out_hbm.at[0, pl.ds(0, CPB * CH)],
                              rows.at[rb], sem).wait()

    def fire_scatters(slot, rb):
        for c in range(CPB):
            pltpu.async_copy(rows.at[rb, pl.ds(c * CH, CH)],
                             acc.at[dblk.at[slot, c]], ssem, add=True)

    load_idx_sync(0, 0)
    fire_gathers(0, 0)
    load_idx_async(1, 1)

    def block(b, carry):
        par = b % 2
        drain_idx((b + 1) % 3)

        @pl.when(b >= 1)
        def _():
            drain_bulk(ssem, 1 - par)

        fire_gathers((b + 1) % 3, 1 - par)
        drain_bulk(gsem, par)
        fire_scatters(b % 3, par)
        load_idx_async(b + 2, (b + 2) % 3)
        return carry

    lax.fori_loop(0, NB - 1, block, 0)

    lpar = (NB - 1) % 2
    drain_idx(NB % 3)
    drain_bulk(ssem, 1 - lpar)
    drain_bulk(gsem, lpar)
    fire_scatters((NB - 1) % 3, lpar)
    drain_bulk(ssem, lpar)

    def tail_chunk(chunk):
        off = chunk * CH
        pltpu.sync_copy(e_hbm.at[0, pl.ds(off, CH)], sflat.at[0, pl.ds(0, CH)])
        pltpu.sync_copy(e_hbm.at[1, pl.ds(off, CH)], dblk.at[0, 0])
        pltpu.async_copy(z_hbm.at[sflat.at[0, pl.ds(0, CH)]],
                         rows.at[0, pl.ds(0, CH)], gsem).wait()
        pltpu.sync_copy(rows.at[0, pl.ds(0, CH)],
                        acc.at[dblk.at[0, 0]], add=True)

    tail_chunk(NW * CH_PW + wid)

    @pl.when(wid < NTAILC - NW)
    def _():
        tail_chunk(NW * CH_PW + NW + wid)

    plsc.subcore_barrier()

    pltpu.sync_copy(acc.at[pl.ds(sid * WOUT, WOUT)],
                    out_hbm.at[cid, pl.ds(sid * WOUT, WOUT)])


def _sc_scatter(z, edge_index, zrows):
    call = pl.kernel(
        _sc_body,
        out_type=jax.ShapeDtypeStruct((NC, NPAD, OUT_DIM), jnp.float32),
        mesh=plsc.VectorSubcoreMesh(core_axis_name="c", subcore_axis_name="s",
                                    num_cores=NC, num_subcores=NS),
        scratch_types=[
            pltpu.VMEM_SHARED((ACC_ROWS, OUT_DIM), jnp.float32),
            pltpu.VMEM((3, CPB * CH), jnp.int32),
            pltpu.VMEM((3, CPB, CH), jnp.int32),
            pltpu.VMEM((2, CPB * CH, OUT_DIM), jnp.float32),
            pltpu.SemaphoreType.DMA,
            pltpu.SemaphoreType.DMA,
            pltpu.SemaphoreType.DMA,
        ],
        compiler_params=pltpu.CompilerParams(use_tc_tiling_on_sc=False),
    )
    return call(z, edge_index, zrows)


def _bcast_lane(vec, lane):
    idx = jnp.full((16, 1), lane, jnp.int32)
    return lax.gather(
        vec, idx,
        dimension_numbers=lax.GatherDimensionNumbers(
            offset_dims=(), collapsed_slice_dims=(0,), start_index_map=(0,)),
        slice_sizes=(1,),
        mode=lax.GatherScatterMode.PROMISE_IN_BOUNDS)


def _fin_body(p_hbm, sn_hbm, out_hbm, vp, sv, ov, lsem, osem):
    cid = lax.axis_index("c")
    sid = lax.axis_index("s")
    wid = sid * NC + cid

    spr = MM_BN // BF

    def fire_loads(j, buf):
        cj = wid + NW * j

        @pl.when(cj < NFULL)
        def _():
            r0 = cj * BF
            pltpu.async_copy(p_hbm.at[:, pl.ds(r0, BF)], vp.at[buf], lsem)
            pltpu.async_copy(sn_hbm.at[cj // spr, 0, pl.ds((cj % spr) * BF, BF)],
                             sv.at[buf], lsem)

    def drain_loads(j, buf):
        cj = wid + NW * j

        @pl.when(cj < NFULL)
        def _():
            pltpu.make_async_copy(p_hbm.at[:, pl.ds(0, BF)],
                                  vp.at[buf], lsem).wait()
            pltpu.make_async_copy(sn_hbm.at[0, 0, pl.ds(0, BF)],
                                  sv.at[buf], lsem).wait()

    def compute(buf, nrows):
        def grp(g, carry):
            s16 = sv[buf, pl.ds(g * 16, 16)]
            for r in range(16):
                row = g * 16 + r
                sr = _bcast_lane(s16, r)
                ov[buf, row, :] = jnp.maximum(
                    vp[buf, 0, row, :] + vp[buf, 1, row, :], 0.0) * sr
            return carry

        lax.fori_loop(0, nrows // 16, grp, 0)

    def drain_store(j, buf):
        cj = wid + NW * j

        @pl.when(cj < NFULL)
        def _():
            pltpu.make_async_copy(ov.at[buf],
                                  out_hbm.at[pl.ds(0, BF)], osem).wait()

    fire_loads(0, 0)

    def loop_body(j, carry):
        buf = j % 2
        fire_loads(j + 1, 1 - buf)
        drain_loads(j, buf)

        @pl.when(j >= 2)
        def _():
            drain_store(j - 2, buf)

        cj = wid + NW * j

        @pl.when(cj < NFULL)
        def _():
            compute(buf, BF)
            pltpu.async_copy(ov.at[buf], out_hbm.at[pl.ds(cj * BF, BF)], osem)

        return carry

    lax.fori_loop(0, CPW_B, loop_body, 0)
    drain_store(CPW_B - 2, CPW_B % 2)
    drain_store(CPW_B - 1, (CPW_B - 1) % 2)


def _finalize(partials, snorm_n):
    call = pl.kernel(
        _fin_body,
        out_type=jax.ShapeDtypeStruct((N, OUT_DIM), jnp.float32),
        mesh=plsc.VectorSubcoreMesh(core_axis_name="c", subcore_axis_name="s",
                                    num_cores=NC, num_subcores=NS),
        scratch_types=[
            pltpu.VMEM((2, 2, BF, OUT_DIM), jnp.float32),
            pltpu.VMEM((2, BF), jnp.float32),
            pltpu.VMEM((2, BF, OUT_DIM), jnp.float32),
            pltpu.SemaphoreType.DMA,
            pltpu.SemaphoreType.DMA,
        ],
        compiler_params=pltpu.CompilerParams(use_tc_tiling_on_sc=False),
    )
    return call(partials, snorm_n)


def kernel(h, edge_index, snorm_n, W_fc, W_attn):
    z = _matmul(h, W_fc.T)
    snf = _snf(snorm_n)
    zrows = jnp.zeros((ROWS_PT, OUT_DIM), jnp.float32)
    partials = _sc_scatter(z, edge_index, zrows)
    return _finalize(partials, snf)


# speedup vs baseline: 100.9492x; 1.0014x over previous
import jax
import jax.numpy as jnp
from jax import lax
from jax.experimental import pallas as pl
from jax.experimental.pallas import tpu as pltpu
from jax.experimental.pallas import tpu_sc as plsc

N = 100000
E = 3200000
IN_DIM = 128
OUT_DIM = 16

NC = 2
NS = 16
NW = NC * NS

CH = 128
CPB = 6
TOTC = E // CH
NB = 130
CH_PW = CPB * NB
NTAILC = TOTC - NW * CH_PW

ACC_ROWS = 100096
ROWS_PT = ACC_ROWS // NS
NPAD = 100096
WOUT = NPAD // NS

MM_BN = 10000

BF = 400
NFULL = N // BF
CPW_B = 8


def _mm_body(h_ref, w_ref, z_ref):
    z_ref[...] = jnp.dot(h_ref[...], w_ref[...],
                         preferred_element_type=jnp.float32)


def _matmul(h, wt):
    return pl.pallas_call(
        _mm_body,
        grid=(N // MM_BN,),
        in_specs=[
            pl.BlockSpec((MM_BN, IN_DIM), lambda i: (i, 0)),
            pl.BlockSpec((IN_DIM, OUT_DIM), lambda i: (0, 0)),
        ],
        out_specs=pl.BlockSpec((MM_BN, OUT_DIM), lambda i: (i, 0)),
        out_shape=jax.ShapeDtypeStruct((N, OUT_DIM), jnp.float32),
    )(h, wt)


def _snf_body(s_ref, o_ref):
    o_ref[...] = s_ref[...].reshape(1, 1, MM_BN)


def _snf(snorm_n):
    nb = N // MM_BN
    return pl.pallas_call(
        _snf_body,
        grid=(nb,),
        in_specs=[pl.BlockSpec((MM_BN, 1), lambda i: (i, 0))],
        out_specs=pl.BlockSpec((1, 1, MM_BN), lambda i: (i, 0, 0)),
        out_shape=jax.ShapeDtypeStruct((nb, 1, MM_BN), jnp.float32),
    )(snorm_n)


def _sc_body(z_hbm, e_hbm, zrows_hbm, out_hbm,
             acc, sflat, dflat, dblk, rows, gsem, ssem, isem):
    cid = lax.axis_index("c")
    sid = lax.axis_index("s")
    wid = sid * NC + cid

    pltpu.sync_copy(zrows_hbm, acc.at[pl.ds(sid * ROWS_PT, ROWS_PT)])
    plsc.subcore_barrier()

    base = wid * CH_PW * CH

    def load_idx(blk, slot, copy_fn):
        off = base + blk * CPB * CH
        copy_fn(e_hbm.at[0, pl.ds(off, CPB * CH)], sflat.at[slot])
        copy_fn(e_hbm.at[1, pl.ds(off, CPB * CH)], dflat.at[slot])

    def load_idx_sync(blk, slot):
        load_idx(blk, slot, pltpu.sync_copy)

    def load_idx_async(blk, slot):
        load_idx(blk, slot,
                 lambda s, d: pltpu.async_copy(s, d, isem))

    def drain_idx(slot):
        for _ in range(2):
            pltpu.make_async_copy(e_hbm.at[0, pl.ds(0, CPB * CH)],
                                  sflat.at[slot], isem).wait()

    def stage_dst(slot):
        for c in range(CPB):
            for g in range(CH // 16):
                dblk[slot, c, pl.ds(g * 16, 16)] = (
                    dflat[slot, pl.ds(c * CH + g * 16, 16)])

    def fire_gathers(slot, rb):
        for c in range(CPB):
            pltpu.async_copy(z_hbm.at[sflat.at[slot, pl.ds(c * CH, CH)]],
                             rows.at[rb, pl.ds(c * CH, CH)], gsem)

    def drain_bulk(sem, rb):
        pltpu.make_async_copy(out_hbm.at[0, pl.ds(0, CPB * CH)],
                              rows.at[rb], sem).wait()

    def fire_scatters(slot, rb):
        for c in range(CPB):
            pltpu.async_copy(rows.at[rb, pl.ds(c * CH, CH)],
                             acc.at[dblk.at[slot, c]], ssem, add=True)

    load_idx_sync(0, 0)
    stage_dst(0)
    fire_gathers(0, 0)
    load_idx_async(1, 1)

    def block(b, carry):
        par = b % 2

        @pl.when(b >= 1)
        def _():
            drain_bulk(ssem, 1 - par)

        drain_idx(1 - par)
        stage_dst(1 - par)
        fire_gathers(1 - par, 1 - par)
        drain_bulk(gsem, par)
        fire_scatters(par, par)
        load_idx_async(b + 2, par)
        return carry

    lax.fori_loop(0, NB - 1, block, 0)

    lpar = (NB - 1) % 2
    drain_idx(1 - lpar)
    drain_bulk(ssem, 1 - lpar)
    drain_bulk(gsem, lpar)
    fire_scatters(lpar, lpar)
    drain_bulk(ssem, lpar)

    def tail_chunk(chunk):
        off = chunk * CH
        pltpu.sync_copy(e_hbm.at[0, pl.ds(off, CH)], sflat.at[0, pl.ds(0, CH)])
        pltpu.sync_copy(e_hbm.at[1, pl.ds(off, CH)], dblk.at[0, 0])
        pltpu.async_copy(z_hbm.at[sflat.at[0, pl.ds(0, CH)]],
                         rows.at[0, pl.ds(0, CH)], gsem).wait()
        pltpu.sync_copy(rows.at[0, pl.ds(0, CH)],
                        acc.at[dblk.at[0, 0]], add=True)

    tail_chunk(NW * CH_PW + wid)

    @pl.when(wid < NTAILC - NW)
    def _():
        tail_chunk(NW * CH_PW + NW + wid)

    plsc.subcore_barrier()

    pltpu.sync_copy(acc.at[pl.ds(sid * WOUT, WOUT)],
                    out_hbm.at[cid, pl.ds(sid * WOUT, WOUT)])


def _sc_scatter(z, edge_index, zrows):
    call = pl.kernel(
        _sc_body,
        out_type=jax.ShapeDtypeStruct((NC, NPAD, OUT_DIM), jnp.float32),
        mesh=plsc.VectorSubcoreMesh(core_axis_name="c", subcore_axis_name="s",
                                    num_cores=NC, num_subcores=NS),
        scratch_types=[
            pltpu.VMEM_SHARED((ACC_ROWS, OUT_DIM), jnp.float32),
            pltpu.VMEM((2, CPB * CH), jnp.int32),
            pltpu.VMEM((2, CPB * CH), jnp.int32),
            pltpu.VMEM((2, CPB, CH), jnp.int32),
            pltpu.VMEM((2, CPB * CH, OUT_DIM), jnp.float32),
            pltpu.SemaphoreType.DMA,
            pltpu.SemaphoreType.DMA,
            pltpu.SemaphoreType.DMA,
        ],
        compiler_params=pltpu.CompilerParams(use_tc_tiling_on_sc=False),
    )
    return call(z, edge_index, zrows)


def _bcast_lane(vec, lane):
    idx = jnp.full((16, 1), lane, jnp.int32)
    return lax.gather(
        vec, idx,
        dimension_numbers=lax.GatherDimensionNumbers(
            offset_dims=(), collapsed_slice_dims=(0,), start_index_map=(0,)),
        slice_sizes=(1,),
        mode=lax.GatherScatterMode.PROMISE_IN_BOUNDS)


def _fin_body(p_hbm, sn_hbm, out_hbm, vp, sv, ov, lsem, osem):
    cid = lax.axis_index("c")
    sid = lax.axis_index("s")
    wid = sid * NC + cid

    spr = MM_BN // BF

    def fire_loads(j, buf):
        cj = wid + NW * j

        @pl.when(cj < NFULL)
        def _():
            r0 = cj * BF
            pltpu.async_copy(p_hbm.at[:, pl.ds(r0, BF)], vp.at[buf], lsem)
            pltpu.async_copy(sn_hbm.at[cj // spr, 0, pl.ds((cj % spr) * BF, BF)],
                             sv.at[buf], lsem)

    def drain_loads(j, buf):
        cj = wid + NW * j

        @pl.when(cj < NFULL)
        def _():
            pltpu.make_async_copy(p_hbm.at[:, pl.ds(0, BF)],
                                  vp.at[buf], lsem).wait()
            pltpu.make_async_copy(sn_hbm.at[0, 0, pl.ds(0, BF)],
                                  sv.at[buf], lsem).wait()

    def compute(buf, nrows):
        def grp(g, carry):
            s16 = sv[buf, pl.ds(g * 16, 16)]
            for r in range(16):
                row = g * 16 + r
                sr = _bcast_lane(s16, r)
                ov[buf, row, :] = jnp.maximum(
                    vp[buf, 0, row, :] + vp[buf, 1, row, :], 0.0) * sr
            return carry

        lax.fori_loop(0, nrows // 16, grp, 0)

    def drain_store(j, buf):
        cj = wid + NW * j

        @pl.when(cj < NFULL)
        def _():
            pltpu.make_async_copy(ov.at[buf],
                                  out_hbm.at[pl.ds(0, BF)], osem).wait()

    fire_loads(0, 0)

    def loop_body(j, carry):
        buf = j % 2
        fire_loads(j + 1, 1 - buf)
        drain_loads(j, buf)

        @pl.when(j >= 2)
        def _():
            drain_store(j - 2, buf)

        cj = wid + NW * j

        @pl.when(cj < NFULL)
        def _():
            compute(buf, BF)
            pltpu.async_copy(ov.at[buf], out_hbm.at[pl.ds(cj * BF, BF)], osem)

        return carry

    lax.fori_loop(0, CPW_B, loop_body, 0)
    drain_store(CPW_B - 2, CPW_B % 2)
    drain_store(CPW_B - 1, (CPW_B - 1) % 2)


def _finalize(partials, snorm_n):
    call = pl.kernel(
        _fin_body,
        out_type=jax.ShapeDtypeStruct((N, OUT_DIM), jnp.float32),
        mesh=plsc.VectorSubcoreMesh(core_axis_name="c", subcore_axis_name="s",
                                    num_cores=NC, num_subcores=NS),
        scratch_types=[
            pltpu.VMEM((2, 2, BF, OUT_DIM), jnp.float32),
            pltpu.VMEM((2, BF), jnp.float32),
            pltpu.VMEM((2, BF, OUT_DIM), jnp.float32),
            pltpu.SemaphoreType.DMA,
            pltpu.SemaphoreType.DMA,
        ],
        compiler_params=pltpu.CompilerParams(use_tc_tiling_on_sc=False),
    )
    return call(partials, snorm_n)


def kernel(h, edge_index, snorm_n, W_fc, W_attn):
    z = _matmul(h, W_fc.T)
    snf = _snf(snorm_n)
    zrows = jnp.zeros((ROWS_PT, OUT_DIM), jnp.float32)
    partials = _sc_scatter(z, edge_index, zrows)
    return _finalize(partials, snf)
